# baseline (speedup 1.0000x reference)
import functools

import jax
import jax.numpy as jnp
from jax import lax
from jax.experimental import pallas as pl
from jax.experimental.pallas import tpu as pltpu
from jax.experimental.pallas import tpu_sc as plsc

_D = 64
_DP = 128
_NW = 32
_T = 2
_SPLITS = ((0, 128), (128, 72))
_VSTARTS = tuple(range(0, 192, 16)) + (184,)


def _build(b, l):
    rows_w = b // _NW
    n_groups = rows_w // _T
    n_pairs = n_groups // 2
    mesh = plsc.VectorSubcoreMesh(core_axis_name="c", subcore_axis_name="s")

    @functools.partial(
        pl.kernel,
        mesh=mesh,
        out_type=jax.ShapeDtypeStruct((b, l, _DP), jnp.float32),
        scratch_types=[
            pltpu.VMEM((2, _T, l), jnp.int32),
            pltpu.VMEM((2, _T, l), jnp.int32),
            pltpu.VMEM((2, _T, l, _DP), jnp.float32),
            pltpu.SemaphoreType.DMA,
            pltpu.SemaphoreType.DMA,
            pltpu.SemaphoreType.DMA,
        ],
        compiler_params=pltpu.CompilerParams(needs_layout_passes=False),
    )
    def emb(idx_hbm, tab_hbm, out_hbm, idx_v, pair_v, rows_v, sem_i, sem_g, sem_o):
        wid = lax.axis_index("s") * 2 + lax.axis_index("c")
        base = wid * rows_w

        def load_idx(g, bf):
            pltpu.async_copy(
                idx_hbm.at[pl.ds(base + g * _T, _T)], idx_v.at[bf], sem_i
            )

        def shift_idx(bf):
            for t in range(_T):
                for s in _VSTARTS:
                    pair_v[bf, t, pl.ds(s, 16)] = (
                        idx_v[bf, t, pl.ds(s, 16)] >> 1
                    )

        def fire(g, bf):
            for t in range(_T):
                for off, n in _SPLITS:
                    pltpu.async_copy(
                        tab_hbm.at[pair_v.at[bf, t, pl.ds(off, n)]],
                        rows_v.at[bf, t, pl.ds(off, n)],
                        sem_g,
                    )

        def fixup(bf):
            for t in range(_T):
                bfv = jnp.full((16,), bf, jnp.int32)
                tv = jnp.full((16,), t, jnp.int32)

                def step(s, pv):
                    jv = lax.iota(jnp.int32, 16) + s
                    for w in range(_D):
                        src = plsc.load_gather(
                            rows_v, [bfv, tv, jv, pv + w]
                        )
                        plsc.store_scatter(
                            rows_v,
                            [bfv, tv, jv, jnp.full((16,), w, jnp.int32)],
                            src,
                        )

                def blk(k, carry):
                    s = k * 16
                    step(s, (idx_v[bf, t, pl.ds(s, 16)] & 1) * _D)
                    return carry

                lax.fori_loop(0, l // 16, blk, 0)
                tail = l - 16
                step(tail, (idx_v[bf, t, pl.ds(tail, 16)] & 1) * _D)

        def drain_rows(bf, sem):
            pltpu.make_async_copy(
                out_hbm.at[pl.ds(0, _T)], rows_v.at[bf], sem
            ).wait()

        def drain_idx(bf):
            pltpu.make_async_copy(
                idx_hbm.at[pl.ds(0, _T)], idx_v.at[bf], sem_i
            ).wait()

        def start_out(g, bf):
            pltpu.async_copy(
                rows_v.at[bf],
                out_hbm.at[pl.ds(base + g * _T, _T)],
                sem_o,
            )

        pltpu.sync_copy(idx_hbm.at[pl.ds(base, _T)], idx_v.at[0])
        shift_idx(0)
        fire(0, 0)
        load_idx(1, 1)

        def pair(p, carry):
            for bf in range(2):
                g = 2 * p + bf
                nb = 1 - bf

                @pl.when(g + 1 < n_groups)
                def _():
                    drain_idx(nb)
                    shift_idx(nb)

                    @pl.when(g >= 1)
                    def _():
                        drain_rows(nb, sem_o)

                    fire(g + 1, nb)

                drain_rows(bf, sem_g)
                fixup(bf)
                start_out(g, bf)

                @pl.when(g + 2 < n_groups)
                def _():
                    load_idx(g + 2, bf)
            return carry

        lax.fori_loop(0, n_pairs, pair, 0)
        drain_rows(0, sem_o)
        drain_rows(1, sem_o)

    return emb


def kernel(token_ids, weight):
    b, l = token_ids.shape
    v, d = weight.shape
    w2 = weight.reshape(v // 2, 2 * d)
    raw = _build(b, l)(token_ids.astype(jnp.int32), w2)
    return raw[:, :, :d]


# speedup vs baseline: 2.1971x; 2.1971x over previous
import functools

import jax
import jax.numpy as jnp
from jax import lax
from jax.experimental import pallas as pl
from jax.experimental.pallas import tpu as pltpu
from jax.experimental.pallas import tpu_sc as plsc

_D = 64
_DP = 128
_NW = 32
_T = 2
_SPLITS = ((0, 128), (128, 72))
_VSTARTS = tuple(range(0, 192, 16)) + (184,)


def _build(b, l):
    rows_w = b // _NW
    n_groups = rows_w // _T
    n_pairs = n_groups // 2
    mesh = plsc.VectorSubcoreMesh(core_axis_name="c", subcore_axis_name="s")

    @functools.partial(
        pl.kernel,
        mesh=mesh,
        out_type=jax.ShapeDtypeStruct((b, l, _DP), jnp.float32),
        scratch_types=[
            pltpu.VMEM((2, _T, l), jnp.int32),
            pltpu.VMEM((2, _T, l), jnp.int32),
            pltpu.VMEM((2, _T, l, _DP), jnp.float32),
            pltpu.SemaphoreType.DMA,
            pltpu.SemaphoreType.DMA,
            pltpu.SemaphoreType.DMA,
        ],
        compiler_params=pltpu.CompilerParams(needs_layout_passes=False),
    )
    def emb(idx_hbm, tab_hbm, out_hbm, idx_v, pair_v, rows_v, sem_i, sem_g, sem_o):
        wid = lax.axis_index("s") * 2 + lax.axis_index("c")
        base = wid * rows_w

        def load_idx(g, bf):
            pltpu.async_copy(
                idx_hbm.at[pl.ds(base + g * _T, _T)], idx_v.at[bf], sem_i
            )

        def shift_idx(bf):
            for t in range(_T):
                for s in _VSTARTS:
                    pair_v[bf, t, pl.ds(s, 16)] = (
                        idx_v[bf, t, pl.ds(s, 16)] >> 1
                    )

        def fire(g, bf):
            for t in range(_T):
                for off, n in _SPLITS:
                    pltpu.async_copy(
                        tab_hbm.at[pair_v.at[bf, t, pl.ds(off, n)]],
                        rows_v.at[bf, t, pl.ds(off, n)],
                        sem_g,
                    )

        def fixup(bf):
            for t in range(_T):

                def step(s, pv):
                    for i in range(16):
                        p = pv[i]
                        for m in range(_D // 16):
                            src = rows_v[bf, t, s + i, pl.ds(p + 16 * m, 16)]
                            rows_v[bf, t, s + i, pl.ds(16 * m, 16)] = src

                def blk(k, carry):
                    s = k * 16
                    step(s, (idx_v[bf, t, pl.ds(s, 16)] & 1) * _D)
                    return carry

                lax.fori_loop(0, l // 16, blk, 0)
                tail = l - 16
                step(tail, (idx_v[bf, t, pl.ds(tail, 16)] & 1) * _D)

        def drain_rows(bf, sem):
            pltpu.make_async_copy(
                out_hbm.at[pl.ds(0, _T)], rows_v.at[bf], sem
            ).wait()

        def drain_idx(bf):
            pltpu.make_async_copy(
                idx_hbm.at[pl.ds(0, _T)], idx_v.at[bf], sem_i
            ).wait()

        def start_out(g, bf):
            pltpu.async_copy(
                rows_v.at[bf],
                out_hbm.at[pl.ds(base + g * _T, _T)],
                sem_o,
            )

        pltpu.sync_copy(idx_hbm.at[pl.ds(base, _T)], idx_v.at[0])
        shift_idx(0)
        fire(0, 0)
        load_idx(1, 1)

        def pair(p, carry):
            for bf in range(2):
                g = 2 * p + bf
                nb = 1 - bf

                @pl.when(g + 1 < n_groups)
                def _():
                    drain_idx(nb)
                    shift_idx(nb)

                    @pl.when(g >= 1)
                    def _():
                        drain_rows(nb, sem_o)

                    fire(g + 1, nb)

                drain_rows(bf, sem_g)
                fixup(bf)
                start_out(g, bf)

                @pl.when(g + 2 < n_groups)
                def _():
                    load_idx(g + 2, bf)
            return carry

        lax.fori_loop(0, n_pairs, pair, 0)
        drain_rows(0, sem_o)
        drain_rows(1, sem_o)

    return emb


def kernel(token_ids, weight):
    b, l = token_ids.shape
    v, d = weight.shape
    w2 = weight.reshape(v // 2, 2 * d)
    raw = _build(b, l)(token_ids.astype(jnp.int32), w2)
    return raw[:, :, :d]


# speedup vs baseline: 2.8125x; 1.2801x over previous
import functools

import jax
import jax.numpy as jnp
from jax import lax
from jax.experimental import pallas as pl
from jax.experimental.pallas import tpu as pltpu
from jax.experimental.pallas import tpu_sc as plsc

_DP = 128
_NW = 32
_T = 2
_SPLITS = ((0, 128), (128, 72))


def _build(b, l):
    rows_w = b // _NW
    n_groups = rows_w // _T
    n_pairs = n_groups // 2
    mesh = plsc.VectorSubcoreMesh(core_axis_name="c", subcore_axis_name="s")

    @functools.partial(
        pl.kernel,
        mesh=mesh,
        out_type=jax.ShapeDtypeStruct((b, l, _DP), jnp.float32),
        scratch_types=[
            pltpu.VMEM((2, _T, l), jnp.int32),
            pltpu.VMEM((2, _T, l, _DP), jnp.float32),
            pltpu.SemaphoreType.DMA,
            pltpu.SemaphoreType.DMA,
            pltpu.SemaphoreType.DMA,
        ],
    )
    def emb(idx_hbm, tab_hbm, out_hbm, idx_v, rows_v, sem_i, sem_g, sem_o):
        wid = lax.axis_index("s") * 2 + lax.axis_index("c")
        base = wid * rows_w

        def load_idx(g, bf):
            pltpu.async_copy(
                idx_hbm.at[pl.ds(base + g * _T, _T)], idx_v.at[bf], sem_i
            )

        def fire(g, bf):
            for t in range(_T):
                for off, n in _SPLITS:
                    pltpu.async_copy(
                        tab_hbm.at[idx_v.at[bf, t, pl.ds(off, n)]],
                        rows_v.at[bf, t, pl.ds(off, n)],
                        sem_g,
                    )

        def drain_rows(bf, sem):
            pltpu.make_async_copy(
                out_hbm.at[pl.ds(0, _T)], rows_v.at[bf], sem
            ).wait()

        def drain_idx(bf):
            pltpu.make_async_copy(
                idx_hbm.at[pl.ds(0, _T)], idx_v.at[bf], sem_i
            ).wait()

        def start_out(g, bf):
            pltpu.async_copy(
                rows_v.at[bf],
                out_hbm.at[pl.ds(base + g * _T, _T)],
                sem_o,
            )

        pltpu.sync_copy(idx_hbm.at[pl.ds(base, _T)], idx_v.at[0])
        fire(0, 0)
        load_idx(1, 1)

        def pair(p, carry):
            for bf in range(2):
                g = 2 * p + bf
                nb = 1 - bf

                @pl.when(g + 1 < n_groups)
                def _():
                    drain_idx(nb)

                    @pl.when(g >= 1)
                    def _():
                        drain_rows(nb, sem_o)

                    fire(g + 1, nb)

                drain_rows(bf, sem_g)
                start_out(g, bf)

                @pl.when(g + 2 < n_groups)
                def _():
                    load_idx(g + 2, bf)
            return carry

        lax.fori_loop(0, n_pairs, pair, 0)
        drain_rows(0, sem_o)
        drain_rows(1, sem_o)

    return emb


def kernel(token_ids, weight):
    b, l = token_ids.shape
    d = weight.shape[1]
    wpad = jnp.pad(weight, ((0, 0), (0, _DP - d)))
    raw = _build(b, l)(token_ids.astype(jnp.int32), wpad)
    return raw[:, :, :d]
